# baseline (speedup 1.0000x reference)
import functools

import jax
import jax.numpy as jnp
from jax import lax
from jax.experimental import pallas as pl
from jax.experimental.pallas import tpu as pltpu
from jax.experimental.pallas import tpu_sc as plsc

_N_EMBED = 1024
_DIM = 256
_B = 8
_HW = 1024
_TOKENS = _B * _HW
_BETA = 0.25
_T_TILE = 512
_N_TILES = _TOKENS // _T_TILE


def _dist_argmin_body(z_ref, e2_ref, bb_ref, idx_ref, md_ref):
    zb = z_ref[...]
    e2 = e2_ref[...]
    bb = bb_ref[...]
    s2 = lax.dot_general(zb, e2, (((1,), (1,)), ((), ())),
                         preferred_element_type=jnp.float32)
    a = jnp.sum(zb * zb, axis=1, keepdims=True)
    d = (a + bb) - s2
    m8 = d[:, 0:128]
    for i in range(1, 8):
        m8 = jnp.minimum(m8, d[:, i * 128:(i + 1) * 128])
    m = jnp.min(m8, axis=1)
    kk = lax.broadcasted_iota(jnp.int32, d.shape, 1).astype(jnp.float32)
    masked = jnp.where(d == m[:, None], kk, jnp.float32(2 ** 30))
    i8 = masked[:, 0:128]
    for i in range(1, 8):
        i8 = jnp.minimum(i8, masked[:, i * 128:(i + 1) * 128])
    idxf = jnp.min(i8, axis=1)
    idx_ref[0, 0, :] = idxf.astype(jnp.int32)
    md_ref[0, 0, :] = m


def _dist_argmin(z2, e2, bb, half):
    nt = _N_TILES // 2
    off = half * nt
    return pl.pallas_call(
        _dist_argmin_body,
        grid=(nt,),
        in_specs=[
            pl.BlockSpec((_T_TILE, _DIM), lambda i: (i + off, 0)),
            pl.BlockSpec((_N_EMBED, _DIM), lambda i: (0, 0)),
            pl.BlockSpec((1, _N_EMBED), lambda i: (0, 0)),
        ],
        out_specs=[
            pl.BlockSpec((1, 1, _T_TILE), lambda i: (i, 0, 0)),
            pl.BlockSpec((1, 1, _T_TILE), lambda i: (i, 0, 0)),
        ],
        out_shape=[
            jax.ShapeDtypeStruct((nt, 1, _T_TILE), jnp.int32),
            jax.ShapeDtypeStruct((nt, 1, _T_TILE), jnp.float32),
        ],
    )(z2, e2, bb)


@functools.lru_cache(maxsize=None)
def _make_sc_gather(half):
    mesh = plsc.VectorSubcoreMesh(core_axis_name="c", subcore_axis_name="s")
    n_half = _TOKENS // 2
    t_per_w = n_half // 32

    @functools.partial(
        pl.kernel,
        mesh=mesh,
        out_type=(),
        scratch_types=[
            pltpu.VMEM((t_per_w,), jnp.int32),
            pltpu.VMEM((t_per_w, _DIM), jnp.float32),
            pltpu.SemaphoreType.DMA,
        ],
    )
    def sck(emb_hbm, idx_hbm, out_hbm, idx_v, rows_v, sem):
        wid = lax.axis_index("s") * 2 + lax.axis_index("c")
        base = wid * t_per_w
        pltpu.sync_copy(idx_hbm.at[pl.ds(base, t_per_w)], idx_v)
        pltpu.async_copy(emb_hbm.at[idx_v], rows_v, sem).wait()
        pltpu.sync_copy(
            rows_v, out_hbm.at[pl.ds(half * n_half + base, t_per_w)])

    return sck


def kernel(z, embedding):
    zp2 = jnp.transpose(z, (0, 2, 3, 1)).reshape(_TOKENS, _DIM)
    e2 = embedding + embedding
    bb = jnp.sum(embedding * embedding, axis=1).reshape(1, _N_EMBED)
    out_ref = jax.new_ref(jnp.zeros((_TOKENS, _DIM), jnp.float32))
    idx_a, md_a = _dist_argmin(zp2, e2, bb, 0)
    _make_sc_gather(0)(embedding, idx_a.reshape(_TOKENS // 2), out_ref)
    idx_b, md_b = _dist_argmin(zp2, e2, bb, 1)
    _make_sc_gather(1)(embedding, idx_b.reshape(_TOKENS // 2), out_ref)
    zq = out_ref[...]
    z_q_out = jnp.transpose(zq.reshape(_B, 32, 32, _DIM), (0, 3, 1, 2))
    loss = _BETA * ((jnp.sum(md_a) + jnp.sum(md_b))
                    / jnp.float32(_TOKENS * _DIM))
    return (z_q_out, loss)


# speedup vs baseline: 1.0763x; 1.0763x over previous
import functools

import jax
import jax.numpy as jnp
from jax import lax
from jax.experimental import pallas as pl
from jax.experimental.pallas import tpu as pltpu
from jax.experimental.pallas import tpu_sc as plsc

_N_EMBED = 1024
_DIM = 256
_B = 8
_HW = 1024
_TOKENS = _B * _HW
_BETA = 0.25
_T_TILE = 512
_N_TILES = _TOKENS // _T_TILE


def _dist_argmin_body(z_ref, e2_ref, bb_ref, idx_ref, md_ref):
    zb = z_ref[...]
    e2 = e2_ref[...]
    bb = bb_ref[...]
    s2 = lax.dot_general(zb, e2, (((1,), (1,)), ((), ())),
                         preferred_element_type=jnp.float32)
    a = jnp.sum(zb * zb, axis=1, keepdims=True)
    d = (a + bb) - s2
    m8 = d[:, 0:128]
    for i in range(1, 8):
        m8 = jnp.minimum(m8, d[:, i * 128:(i + 1) * 128])
    m = jnp.min(m8, axis=1)
    kk = lax.broadcasted_iota(jnp.int32, d.shape, 1).astype(jnp.float32)
    masked = jnp.where(d == m[:, None], kk, jnp.float32(2 ** 30))
    i8 = masked[:, 0:128]
    for i in range(1, 8):
        i8 = jnp.minimum(i8, masked[:, i * 128:(i + 1) * 128])
    idxf = jnp.min(i8, axis=1)
    idx_ref[0, 0, :] = idxf.astype(jnp.int32)
    md_ref[0, 0, :] = m


def _dist_argmin(z2, e2, bb):
    return pl.pallas_call(
        _dist_argmin_body,
        grid=(_N_TILES,),
        compiler_params=pltpu.CompilerParams(
            allow_input_fusion=[True, True, True]),
        in_specs=[
            pl.BlockSpec((_T_TILE, _DIM), lambda i: (i, 0)),
            pl.BlockSpec((_N_EMBED, _DIM), lambda i: (0, 0)),
            pl.BlockSpec((1, _N_EMBED), lambda i: (0, 0)),
        ],
        out_specs=[
            pl.BlockSpec((1, 1, _T_TILE), lambda i: (i, 0, 0)),
            pl.BlockSpec((1, 1, _T_TILE), lambda i: (i, 0, 0)),
        ],
        out_shape=[
            jax.ShapeDtypeStruct((_N_TILES, 1, _T_TILE), jnp.int32),
            jax.ShapeDtypeStruct((_N_TILES, 1, _T_TILE), jnp.float32),
        ],
    )(z2, e2, bb)


@functools.lru_cache(maxsize=None)
def _make_sc_gather():
    mesh = plsc.VectorSubcoreMesh(core_axis_name="c", subcore_axis_name="s")
    t_per_w = _TOKENS // 32

    @functools.partial(
        pl.kernel,
        mesh=mesh,
        out_type=jax.ShapeDtypeStruct((_TOKENS, _DIM), jnp.float32),
        scratch_types=[
            pltpu.VMEM((t_per_w,), jnp.int32),
            pltpu.VMEM((t_per_w, _DIM), jnp.float32),
            pltpu.SemaphoreType.DMA,
        ],
    )
    def sck(emb_hbm, idx_hbm, out_hbm, idx_v, rows_v, sem):
        wid = lax.axis_index("s") * 2 + lax.axis_index("c")
        base = wid * t_per_w
        pltpu.sync_copy(idx_hbm.at[pl.ds(base, t_per_w)], idx_v)
        pltpu.async_copy(emb_hbm.at[idx_v], rows_v, sem).wait()
        pltpu.sync_copy(rows_v, out_hbm.at[pl.ds(base, t_per_w)])

    return sck


def kernel(z, embedding):
    zp2 = jnp.transpose(z, (0, 2, 3, 1)).reshape(_TOKENS, _DIM)
    e2 = embedding + embedding
    bb = jnp.sum(embedding * embedding, axis=1).reshape(1, _N_EMBED)
    idx3, md3 = _dist_argmin(zp2, e2, bb)
    zq = _make_sc_gather()(embedding, idx3.reshape(_TOKENS))
    z_q_out = jnp.transpose(zq.reshape(_B, 32, 32, _DIM), (0, 3, 1, 2))
    loss = _BETA * (jnp.sum(md3) / jnp.float32(_TOKENS * _DIM))
    return (z_q_out, loss)
